# baseline (speedup 1.0000x reference)
from functools import partial

import numpy as np
import jax
import jax.numpy as jnp
from jax.experimental import pallas as pl
from jax.experimental.pallas import tpu as pltpu

IMG = 28
KS = 5
C1, C2 = 10, 20
F1, F2 = 50, 10
P1 = 12
P2 = 4
N1SLAB = C1 * P1 * P1
N2SLAB = C2 * P2 * P2
NB1 = 6
K1 = 224
N1B = 240


def _c1_stack():
    C = np.zeros((4, NB1, K1, 24, 25), np.float32)
    for di in (0, 1):
        for dj in (0, 1):
            ph = di * 2 + dj
            for G in range(NB1):
                for i in range(KS):
                    for j in range(KS):
                        t = i * KS + j
                        for al in (0, 1):
                            for e in range(P1):
                                p = (2 * al + di + i) * IMG + (2 * e + dj + j)
                                C[ph, G, p, al * P1 + e, t] = 1.0
    return C.reshape(4 * NB1 * K1 * 24, 25)


def _c2cat():
    out = []
    for di in (0, 1):
        for dj in (0, 1):
            D = np.zeros((P1 * P1, 25 * 16), np.float32)
            for i in range(KS):
                for j in range(KS):
                    t = i * KS + j
                    for a in range(P2):
                        for e in range(P2):
                            uv = (2 * a + di + i) * P1 + (2 * e + dj + j)
                            D[uv, t * 16 + a * P2 + e] = 1.0
            out.append(D)
    return out


_CONSTS = None


def _get_consts():
    global _CONSTS
    if _CONSTS is None:
        _CONSTS = (_c1_stack(), _c2cat())
    return _CONSTS


def _build_m1_stack(w1):
    c1, _ = _get_consts()
    out = jnp.dot(jnp.asarray(c1, jnp.bfloat16), w1.T.astype(jnp.bfloat16),
                  preferred_element_type=jnp.float32)
    return out.reshape(4 * NB1, K1, N1B).astype(jnp.bfloat16)


def _build_m2_blocks(w2):
    _, c2s = _get_consts()
    w2r = w2.reshape(C2, 25, C1).transpose(2, 1, 0).astype(jnp.bfloat16)
    eye16 = jnp.asarray(np.eye(16, dtype=np.float32), jnp.bfloat16)
    kw = (w2r[:, :, None, :, None] * eye16[None, None, :, None, :]
          ).reshape(C1, 25 * 16, C2 * 16)
    blocks = []
    for d in c2s:
        lhs = jnp.broadcast_to(jnp.asarray(d, jnp.bfloat16)[None],
                               (C1, 144, 25 * 16))
        blk = jax.lax.dot_general(
            lhs, kw, (((2,), (1,)), ((0,), (0,))),
            preferred_element_type=jnp.float32)
        blocks.append(jnp.transpose(blk, (1, 0, 2))
                      .reshape(N1SLAB, N2SLAB).astype(jnp.bfloat16))
    return blocks


def _net_kernel(x_ref, m1_ref, b1b_ref,
                m2a_ref, m2b_ref, m2c_ref, m2d_ref, b2s_ref,
                wf1_ref, bf1_ref, wf2_ref, bf2_ref, o_ref):
    xb = x_ref[...]
    b1b = b1b_ref[...].astype(jnp.bfloat16)

    parts = []
    for g in range(NB1):
        xs = xb[:, 112 * g:112 * g + K1]
        ya = jnp.dot(xs, m1_ref[0 * NB1 + g],
                     preferred_element_type=jnp.float32).astype(jnp.bfloat16)
        yb = jnp.dot(xs, m1_ref[1 * NB1 + g],
                     preferred_element_type=jnp.float32).astype(jnp.bfloat16)
        yc = jnp.dot(xs, m1_ref[2 * NB1 + g],
                     preferred_element_type=jnp.float32).astype(jnp.bfloat16)
        yd = jnp.dot(xs, m1_ref[3 * NB1 + g],
                     preferred_element_type=jnp.float32).astype(jnp.bfloat16)
        t1 = jnp.maximum(jnp.maximum(ya, yb), jnp.maximum(yc, yd))
        parts.append(jnp.maximum(t1 + b1b, jnp.bfloat16(0.0)))
    p1 = jnp.concatenate(parts, axis=1)

    y2a = jnp.dot(p1, m2a_ref[...],
                  preferred_element_type=jnp.float32).astype(jnp.bfloat16)
    y2b = jnp.dot(p1, m2b_ref[...],
                  preferred_element_type=jnp.float32).astype(jnp.bfloat16)
    y2c = jnp.dot(p1, m2c_ref[...],
                  preferred_element_type=jnp.float32).astype(jnp.bfloat16)
    y2d = jnp.dot(p1, m2d_ref[...],
                  preferred_element_type=jnp.float32).astype(jnp.bfloat16)
    t2 = jnp.maximum(jnp.maximum(y2a, y2b), jnp.maximum(y2c, y2d))
    flat = jnp.maximum(t2 + b2s_ref[...].astype(jnp.bfloat16),
                       jnp.bfloat16(0.0))

    h = jnp.dot(flat, wf1_ref[...],
                preferred_element_type=jnp.float32) + bf1_ref[...]
    h = jnp.maximum(h, 0.0)

    logits = jnp.dot(h.astype(jnp.bfloat16), wf2_ref[...],
                     preferred_element_type=jnp.float32) + bf2_ref[...]
    logits = logits - jnp.max(logits, axis=-1, keepdims=True)
    out = logits - jnp.log(jnp.sum(jnp.exp(logits), axis=-1, keepdims=True))
    o_ref[...] = out.astype(o_ref.dtype)


@partial(jax.jit, static_argnames=("block_batch",))
def _forward(x, w1, b1, w2, b2, wf1, bf1, wf2, bf2, block_batch=1024):
    n = x.shape[0]
    bb = min(block_batch, max(8, ((n + 7) // 8) * 8))
    n_pad = ((n + bb - 1) // bb) * bb

    xf = x.reshape(n, IMG * IMG).astype(jnp.bfloat16)
    if n_pad != n:
        xf = jnp.pad(xf, ((0, n_pad - n), (0, 0)))

    m1 = _build_m1_stack(w1)
    m2s = _build_m2_blocks(w2)
    b1b = jnp.broadcast_to(b1.reshape(1, C1), (24, C1)).reshape(1, N1B)
    b2s = jnp.broadcast_to(b2.reshape(C2, 1), (C2, P2 * P2)).reshape(1, N2SLAB)

    flops_per = 2 * (4 * NB1 * K1 * N1B + 4 * N1SLAB * N2SLAB
                     + N2SLAB * F1 + F1 * F2)
    ce = pl.CostEstimate(
        flops=n_pad * flops_per,
        transcendentals=n_pad * (F2 + 1),
        bytes_accessed=n_pad * (IMG * IMG * 2 + F2 * 4)
        + 2 * 24 * K1 * N1B + 8 * N1SLAB * N2SLAB)

    w2cols = [pl.BlockSpec((N1SLAB, N2SLAB), lambda g: (0, 0))] * 4
    out = pl.pallas_call(
        _net_kernel,
        out_shape=jax.ShapeDtypeStruct((n_pad, F2), jnp.float32),
        grid=(n_pad // bb,),
        in_specs=[
            pl.BlockSpec((bb, IMG * IMG), lambda g: (g, 0)),
            pl.BlockSpec((4 * NB1, K1, N1B), lambda g: (0, 0, 0)),
            pl.BlockSpec((1, N1B), lambda g: (0, 0)),
            *w2cols,
            pl.BlockSpec((1, N2SLAB), lambda g: (0, 0)),
            pl.BlockSpec((N2SLAB, F1), lambda g: (0, 0)),
            pl.BlockSpec((1, F1), lambda g: (0, 0)),
            pl.BlockSpec((F1, F2), lambda g: (0, 0)),
            pl.BlockSpec((1, F2), lambda g: (0, 0)),
        ],
        out_specs=pl.BlockSpec((bb, F2), lambda g: (g, 0)),
        compiler_params=pltpu.CompilerParams(
            dimension_semantics=("parallel",)),
        cost_estimate=ce,
    )(xf, m1, b1b, *m2s, b2s, wf1, bf1, wf2, bf2)
    return out[:n]


def kernel(x, w1, b1, w2, b2, s2, ssp, mflat, wf1, bf1, wf2, bf2):
    del s2, ssp, mflat
    return _forward(x, w1, b1, w2, b2, wf1, bf1, wf2, bf2)
